# Initial kernel scaffold; baseline (speedup 1.0000x reference)
import jax
import jax.numpy as jnp
from jax.experimental import pallas as pl


def kernel(conditioning, W_w, b_w, W_h, b_h, W_d, b_d, W_l, b_l):
    raise NotImplementedError("write your pallas kernel here")


# speedup vs baseline: 1.0509x; 1.0509x over previous
import jax
import jax.numpy as jnp
from jax.experimental import pallas as pl
from jax.experimental.pallas import tpu as pltpu

D_MODEL = 768
N_PAD = 64
BLOCK_M = 2048


def _fused_heads_kernel(x_ref, w_ref, b_ref, o_ref):
    o_ref[...] = (
        jnp.dot(x_ref[...], w_ref[...], preferred_element_type=jnp.float32)
        + b_ref[...]
    )


def kernel(conditioning, W_w, b_w, W_h, b_h, W_d, b_d, W_l, b_l):
    B, T, D = conditioning.shape
    M = B * T
    x = conditioning.reshape(M, D)

    W_cat = jnp.concatenate([W_w, W_h, W_d, W_l], axis=0)
    W_cat = jnp.pad(W_cat, ((0, N_PAD - W_cat.shape[0]), (0, 0))).T
    b_cat = jnp.concatenate([b_w, b_h, b_d, b_l], axis=0)
    b_cat = jnp.pad(b_cat, (0, N_PAD - b_cat.shape[0])).reshape(1, N_PAD)

    grid = (M // BLOCK_M,)
    out = pl.pallas_call(
        _fused_heads_kernel,
        grid=grid,
        in_specs=[
            pl.BlockSpec((BLOCK_M, D), lambda i: (i, 0)),
            pl.BlockSpec((D, N_PAD), lambda i: (0, 0)),
            pl.BlockSpec((1, N_PAD), lambda i: (0, 0)),
        ],
        out_specs=pl.BlockSpec((BLOCK_M, N_PAD), lambda i: (i, 0)),
        out_shape=jax.ShapeDtypeStruct((M, N_PAD), jnp.float32),
        compiler_params=pltpu.CompilerParams(
            dimension_semantics=("arbitrary",),
        ),
    )(x, W_cat, b_cat)

    widths = out[:, 0:16].reshape(B, T, 16)
    heights = out[:, 16:32].reshape(B, T, 16)
    derivatives = out[:, 32:47].reshape(B, T, 15)
    lambdas = out[:, 47:63].reshape(B, T, 16)
    return (widths, heights, derivatives, lambdas)


# speedup vs baseline: 1.3549x; 1.2893x over previous
import jax
import jax.numpy as jnp
from jax.experimental import pallas as pl
from jax.experimental.pallas import tpu as pltpu

D_MODEL = 768
N_PAD = 64
BLOCK_M = 2048


def _fused_heads_kernel(x_ref, w_ref, b_ref, ow_ref, oh_ref, od_ref, ol_ref):
    res = (
        jnp.dot(x_ref[...], w_ref[...], preferred_element_type=jnp.float32)
        + b_ref[...]
    )
    ow_ref[...] = res[:, 0:16]
    oh_ref[...] = res[:, 16:32]
    od_ref[...] = res[:, 32:47]
    ol_ref[...] = res[:, 47:63]


def kernel(conditioning, W_w, b_w, W_h, b_h, W_d, b_d, W_l, b_l):
    B, T, D = conditioning.shape
    M = B * T
    x = conditioning.reshape(M, D)

    W_cat = jnp.concatenate([W_w, W_h, W_d, W_l], axis=0)
    W_cat = jnp.pad(W_cat, ((0, N_PAD - W_cat.shape[0]), (0, 0))).T
    b_cat = jnp.concatenate([b_w, b_h, b_d, b_l], axis=0)
    b_cat = jnp.pad(b_cat, (0, N_PAD - b_cat.shape[0])).reshape(1, N_PAD)

    grid = (M // BLOCK_M,)
    ow, oh, od, ol = pl.pallas_call(
        _fused_heads_kernel,
        grid=grid,
        in_specs=[
            pl.BlockSpec((BLOCK_M, D), lambda i: (i, 0)),
            pl.BlockSpec((D, N_PAD), lambda i: (0, 0)),
            pl.BlockSpec((1, N_PAD), lambda i: (0, 0)),
        ],
        out_specs=[
            pl.BlockSpec((BLOCK_M, 16), lambda i: (i, 0)),
            pl.BlockSpec((BLOCK_M, 16), lambda i: (i, 0)),
            pl.BlockSpec((BLOCK_M, 15), lambda i: (i, 0)),
            pl.BlockSpec((BLOCK_M, 16), lambda i: (i, 0)),
        ],
        out_shape=[
            jax.ShapeDtypeStruct((M, 16), jnp.float32),
            jax.ShapeDtypeStruct((M, 16), jnp.float32),
            jax.ShapeDtypeStruct((M, 15), jnp.float32),
            jax.ShapeDtypeStruct((M, 16), jnp.float32),
        ],
        compiler_params=pltpu.CompilerParams(
            dimension_semantics=("arbitrary",),
        ),
    )(x, W_cat, b_cat)

    return (
        ow.reshape(B, T, 16),
        oh.reshape(B, T, 16),
        od.reshape(B, T, 15),
        ol.reshape(B, T, 16),
    )


# speedup vs baseline: 1.3575x; 1.0019x over previous
import jax
import jax.numpy as jnp
from jax.experimental import pallas as pl
from jax.experimental.pallas import tpu as pltpu

D_MODEL = 768
N_PAD = 64
BLOCK_M = 2048


def _fused_heads_kernel(x_ref, w_ref, b_ref, ow_ref, oh_ref, od_ref, ol_ref):
    res = (
        jnp.dot(x_ref[...], w_ref[...], preferred_element_type=jnp.float32)
        + b_ref[...]
    )
    ow_ref[...] = res[:, 0:16]
    oh_ref[...] = res[:, 16:32]
    od_ref[...] = res[:, 32:47]
    ol_ref[...] = res[:, 47:63]


def kernel(conditioning, W_w, b_w, W_h, b_h, W_d, b_d, W_l, b_l):
    B, T, D = conditioning.shape
    M = B * T
    x = conditioning.reshape(M, D)

    W_cat = jnp.concatenate([W_w, W_h, W_d, W_l], axis=0)
    W_cat = jnp.pad(W_cat, ((0, N_PAD - W_cat.shape[0]), (0, 0))).T
    b_cat = jnp.concatenate([b_w, b_h, b_d, b_l], axis=0)
    b_cat = jnp.pad(b_cat, (0, N_PAD - b_cat.shape[0])).reshape(1, N_PAD)

    grid = (M // BLOCK_M,)
    ow, oh, od, ol = pl.pallas_call(
        _fused_heads_kernel,
        grid=grid,
        in_specs=[
            pl.BlockSpec((BLOCK_M, D), lambda i: (i, 0)),
            pl.BlockSpec((D, N_PAD), lambda i: (0, 0)),
            pl.BlockSpec((1, N_PAD), lambda i: (0, 0)),
        ],
        out_specs=[
            pl.BlockSpec((BLOCK_M, 16), lambda i: (i, 0)),
            pl.BlockSpec((BLOCK_M, 16), lambda i: (i, 0)),
            pl.BlockSpec((BLOCK_M, 15), lambda i: (i, 0)),
            pl.BlockSpec((BLOCK_M, 16), lambda i: (i, 0)),
        ],
        out_shape=[
            jax.ShapeDtypeStruct((M, 16), jnp.float32),
            jax.ShapeDtypeStruct((M, 16), jnp.float32),
            jax.ShapeDtypeStruct((M, 15), jnp.float32),
            jax.ShapeDtypeStruct((M, 16), jnp.float32),
        ],
        compiler_params=pltpu.CompilerParams(
            dimension_semantics=("parallel",),
        ),
    )(x, W_cat, b_cat)

    return (
        ow.reshape(B, T, 16),
        oh.reshape(B, T, 16),
        od.reshape(B, T, 15),
        ol.reshape(B, T, 16),
    )


# speedup vs baseline: 1.3770x; 1.0143x over previous
import jax
import jax.numpy as jnp
from jax.experimental import pallas as pl
from jax.experimental.pallas import tpu as pltpu

D_MODEL = 768
N_PAD = 64
BLOCK_M = 4096


def _fused_heads_kernel(x_ref, w_ref, b_ref, ow_ref, oh_ref, od_ref, ol_ref):
    res = (
        jnp.dot(x_ref[...], w_ref[...], preferred_element_type=jnp.float32)
        + b_ref[...]
    )
    ow_ref[...] = res[:, 0:16]
    oh_ref[...] = res[:, 16:32]
    od_ref[...] = res[:, 32:47]
    ol_ref[...] = res[:, 47:63]


def kernel(conditioning, W_w, b_w, W_h, b_h, W_d, b_d, W_l, b_l):
    B, T, D = conditioning.shape
    M = B * T
    x = conditioning.reshape(M, D)

    W_cat = jnp.concatenate([W_w, W_h, W_d, W_l], axis=0)
    W_cat = jnp.pad(W_cat, ((0, N_PAD - W_cat.shape[0]), (0, 0))).T
    b_cat = jnp.concatenate([b_w, b_h, b_d, b_l], axis=0)
    b_cat = jnp.pad(b_cat, (0, N_PAD - b_cat.shape[0])).reshape(1, N_PAD)

    grid = (M // BLOCK_M,)
    ow, oh, od, ol = pl.pallas_call(
        _fused_heads_kernel,
        grid=grid,
        in_specs=[
            pl.BlockSpec((BLOCK_M, D), lambda i: (i, 0)),
            pl.BlockSpec((D, N_PAD), lambda i: (0, 0)),
            pl.BlockSpec((1, N_PAD), lambda i: (0, 0)),
        ],
        out_specs=[
            pl.BlockSpec((BLOCK_M, 16), lambda i: (i, 0)),
            pl.BlockSpec((BLOCK_M, 16), lambda i: (i, 0)),
            pl.BlockSpec((BLOCK_M, 15), lambda i: (i, 0)),
            pl.BlockSpec((BLOCK_M, 16), lambda i: (i, 0)),
        ],
        out_shape=[
            jax.ShapeDtypeStruct((M, 16), jnp.float32),
            jax.ShapeDtypeStruct((M, 16), jnp.float32),
            jax.ShapeDtypeStruct((M, 15), jnp.float32),
            jax.ShapeDtypeStruct((M, 16), jnp.float32),
        ],
        compiler_params=pltpu.CompilerParams(
            dimension_semantics=("parallel",),
        ),
    )(x, W_cat, b_cat)

    return (
        ow.reshape(B, T, 16),
        oh.reshape(B, T, 16),
        od.reshape(B, T, 15),
        ol.reshape(B, T, 16),
    )
